# baseline (speedup 1.0000x reference)
import functools

import jax
import jax.numpy as jnp
from jax import lax
from jax.experimental import pallas as pl
from jax.experimental.pallas import tpu as pltpu
from jax.experimental.pallas import tpu_sc as plsc

NC = 2
NS = 16
NW = NC * NS
EC = 128
EMB_D = 128


def _make_sc_kernel(e_pad: int, num_nodes: int):
    per_w = e_pad // NW
    chunks = per_w // EC
    mesh = plsc.VectorSubcoreMesh(
        core_axis_name="c", subcore_axis_name="s", num_cores=NC, num_subcores=NS
    )

    @functools.partial(
        pl.kernel,
        out_type=jax.ShapeDtypeStruct((e_pad,), jnp.float32),
        mesh=mesh,
        compiler_params=pltpu.CompilerParams(needs_layout_passes=False),
        scratch_types=[
            pltpu.VMEM((EC,), jnp.int32),
            pltpu.VMEM((EC,), jnp.int32),
            pltpu.VMEM((EC,), jnp.int32),
            pltpu.VMEM((EC,), jnp.int32),
            pltpu.VMEM((EC, EMB_D), jnp.float32),
            pltpu.VMEM((EC, EMB_D), jnp.float32),
            pltpu.VMEM((EC,), jnp.float32),
            pltpu.SemaphoreType.DMA,
            pltpu.SemaphoreType.DMA,
        ],
    )
    def sc_kernel(src_h, dst_h, nid_h, emb_h, out_h,
                  sidx, didx, aidx, bidx, arows, brows, outv, sem_a, sem_b):
        wid = lax.axis_index("s") * NC + lax.axis_index("c")
        lane = lax.iota(jnp.int32, 16)
        e_idx = [lane + 16 * g for g in range(EC // 16)]

        @pl.loop(0, chunks)
        def chunk_loop(ci):
            base = wid * per_w + ci * EC
            pltpu.sync_copy(src_h.at[pl.ds(base, EC)], sidx)
            pltpu.sync_copy(dst_h.at[pl.ds(base, EC)], didx)
            cpa = pltpu.async_copy(nid_h.at[sidx], aidx, sem_a)
            cpb = pltpu.async_copy(nid_h.at[didx], bidx, sem_b)
            cpa.wait()
            cpb.wait()
            cra = pltpu.async_copy(emb_h.at[aidx], arows, sem_a)
            crb = pltpu.async_copy(emb_h.at[bidx], brows, sem_b)
            cra.wait()
            crb.wait()

            def dbody(d, accs):
                dsplat = jnp.full((16,), d, jnp.int32)
                out = []
                for g in range(EC // 16):
                    av = plsc.load_gather(arows, [e_idx[g], dsplat])
                    bv = plsc.load_gather(brows, [e_idx[g], dsplat])
                    out.append(accs[g] + av * bv)
                return tuple(out)

            accs0 = tuple(jnp.zeros((16,), jnp.float32) for _ in range(EC // 16))
            accs = lax.fori_loop(0, EMB_D, dbody, accs0)
            for g in range(EC // 16):
                outv[pl.ds(16 * g, 16)] = accs[g]
            pltpu.sync_copy(outv, out_h.at[pl.ds(base, EC)])

    return sc_kernel


def kernel(n_id, edge_label_index, emb):
    num_edges = edge_label_index.shape[1]
    num_nodes = emb.shape[0]
    e_pad = ((num_edges + NW * EC - 1) // (NW * EC)) * (NW * EC)
    src = jnp.pad(edge_label_index[0], (0, e_pad - num_edges))
    dst = jnp.pad(edge_label_index[1], (0, e_pad - num_edges))
    sc = _make_sc_kernel(e_pad, num_nodes)
    out = sc(src, dst, n_id, emb)
    return out[:num_edges]


# speedup vs baseline: 1.2469x; 1.2469x over previous
import functools

import jax
import jax.numpy as jnp
from jax import lax
from jax.experimental import pallas as pl
from jax.experimental.pallas import tpu as pltpu
from jax.experimental.pallas import tpu_sc as plsc

NC = 2
NS = 16
NW = NC * NS
EC = 128
EMB_D = 128
UNROLL = 6


def _make_sc_kernel(e_pad: int):
    per_w = e_pad // NW
    chunks = per_w // EC
    assert chunks % UNROLL == 0
    mesh = plsc.VectorSubcoreMesh(
        core_axis_name="c", subcore_axis_name="s", num_cores=NC, num_subcores=NS
    )

    @functools.partial(
        pl.kernel,
        out_type=jax.ShapeDtypeStruct((e_pad,), jnp.float32),
        mesh=mesh,
        compiler_params=pltpu.CompilerParams(needs_layout_passes=False),
        scratch_types=[
            pltpu.VMEM((per_w,), jnp.int32),
            pltpu.VMEM((per_w,), jnp.int32),
            pltpu.VMEM((3, EC), jnp.int32),
            pltpu.VMEM((3, EC), jnp.int32),
            pltpu.VMEM((2, EC, EMB_D), jnp.float32),
            pltpu.VMEM((2, EC, EMB_D), jnp.float32),
            pltpu.VMEM((2, EC), jnp.float32),
            pltpu.SemaphoreType.DMA((2,)),
            pltpu.SemaphoreType.DMA((2,)),
            pltpu.SemaphoreType.DMA((2,)),
        ],
    )
    def sc_kernel(src_h, dst_h, nid_h, emb_h, out_h,
                  sall, dall, aidx, bidx, arows, brows, outv,
                  sem_n, sem_r, sem_o):
        wid = lax.axis_index("s") * NC + lax.axis_index("c")
        wbase = wid * per_w
        lane = lax.iota(jnp.int32, 16)
        e_idx = [lane + 16 * g for g in range(EC // 16)]

        pltpu.sync_copy(src_h.at[pl.ds(wbase, per_w)], sall)
        pltpu.sync_copy(dst_h.at[pl.ds(wbase, per_w)], dall)

        def issue_nid(c, islot, sslot):
            pltpu.async_copy(nid_h.at[sall.at[pl.ds(c * EC, EC)]],
                             aidx.at[islot], sem_n.at[sslot])
            pltpu.async_copy(nid_h.at[dall.at[pl.ds(c * EC, EC)]],
                             bidx.at[islot], sem_n.at[sslot])

        def wait_nid(c, islot, sslot):
            pltpu.make_async_copy(nid_h.at[sall.at[pl.ds(c * EC, EC)]],
                                  aidx.at[islot], sem_n.at[sslot]).wait()
            pltpu.make_async_copy(nid_h.at[dall.at[pl.ds(c * EC, EC)]],
                                  bidx.at[islot], sem_n.at[sslot]).wait()

        def issue_rows(islot, rslot):
            pltpu.async_copy(emb_h.at[aidx.at[islot]], arows.at[rslot],
                             sem_r.at[rslot])
            pltpu.async_copy(emb_h.at[bidx.at[islot]], brows.at[rslot],
                             sem_r.at[rslot])

        def wait_rows(islot, rslot):
            pltpu.make_async_copy(emb_h.at[aidx.at[islot]], arows.at[rslot],
                                  sem_r.at[rslot]).wait()
            pltpu.make_async_copy(emb_h.at[bidx.at[islot]], brows.at[rslot],
                                  sem_r.at[rslot]).wait()

        def issue_out(c, oslot):
            pltpu.async_copy(outv.at[oslot], out_h.at[pl.ds(wbase + c * EC, EC)],
                             sem_o.at[oslot])

        def wait_out(c, oslot):
            pltpu.make_async_copy(outv.at[oslot],
                                  out_h.at[pl.ds(wbase + c * EC, EC)],
                                  sem_o.at[oslot]).wait()

        def compute(rslot, oslot):
            ar = arows.at[rslot]
            br = brows.at[rslot]

            @plsc.parallel_loop(
                0, EMB_D, unroll=4,
                carry=tuple(jnp.zeros((16,), jnp.float32)
                            for _ in range(EC // 16)),
            )
            def accs(d, carry):
                dsplat = jnp.full((16,), d, jnp.int32)
                out = []
                for g in range(EC // 16):
                    av = plsc.load_gather(ar, [e_idx[g], dsplat])
                    bv = plsc.load_gather(br, [e_idx[g], dsplat])
                    out.append(carry[g] + av * bv)
                return tuple(out)

            ov = outv.at[oslot]
            for g in range(EC // 16):
                ov[pl.ds(16 * g, 16)] = accs[g]

        issue_nid(0, 0, 0)
        issue_nid(1, 1, 1)
        wait_nid(0, 0, 0)
        issue_rows(0, 0)

        @pl.loop(0, chunks // UNROLL)
        def chunk_loop(jo):
            for b in range(UNROLL):
                j = jo * UNROLL + b
                @pl.when(j + 2 < chunks)
                def _():
                    issue_nid(j + 2, (b + 2) % 3, b % 2)

                @pl.when(j + 1 < chunks)
                def _():
                    wait_nid(j + 1, (b + 1) % 3, (b + 1) % 2)
                    issue_rows((b + 1) % 3, (b + 1) % 2)

                wait_rows(b % 3, b % 2)

                @pl.when(j >= 2)
                def _():
                    wait_out(j - 2, b % 2)

                compute(b % 2, b % 2)
                issue_out(j, b % 2)

        wait_out(chunks - 2, (chunks - 2) % 2)
        wait_out(chunks - 1, (chunks - 1) % 2)

    return sc_kernel


def kernel(n_id, edge_label_index, emb):
    num_edges = edge_label_index.shape[1]
    grain = NW * EC * UNROLL
    e_pad = ((num_edges + grain - 1) // grain) * grain
    src = jnp.pad(edge_label_index[0], (0, e_pad - num_edges))
    dst = jnp.pad(edge_label_index[1], (0, e_pad - num_edges))
    sc = _make_sc_kernel(e_pad)
    out = sc(src, dst, n_id, emb)
    return out[:num_edges]
